# baseline (speedup 1.0000x reference)
import jax
import jax.numpy as jnp
from jax.experimental import pallas as pl
from jax.experimental.pallas import tpu as pltpu

_BIG = 1 << 21
_SMALL = 1 << 19
_SIZES = [_SMALL, _SMALL, _BIG, _BIG, _BIG, _BIG, _BIG, _BIG, _BIG,
          _SMALL, _SMALL - 128]
assert sum(_SIZES) == 128 * 131071
_NBUF = 4


def _shift_body(data_ref, x_ref, out_ref, bufs, lsems, ssems, hsem):
    nc = len(_SIZES)
    offs = [0]
    for s in _SIZES:
        offs.append(offs[-1] + s)

    def load(k):
        b = k % _NBUF
        return pltpu.make_async_copy(
            data_ref.at[pl.ds(offs[k], _SIZES[k])],
            bufs.at[b, pl.ds(0, _SIZES[k])],
            lsems.at[b],
        )

    def store(k):
        b = k % _NBUF
        return pltpu.make_async_copy(
            bufs.at[b, pl.ds(0, _SIZES[k])],
            out_ref.at[pl.ds(128 + offs[k], _SIZES[k])],
            ssems.at[b],
        )

    loads = [load(k) for k in range(nc)]
    stores = [store(k) for k in range(nc)]

    head = pltpu.make_async_copy(x_ref, out_ref.at[pl.ds(0, 128)], hsem)
    head.start()

    for k in range(min(_NBUF, nc)):
        loads[k].start(priority=k % 2)
    for k in range(nc):
        loads[k].wait()
        stores[k].start(priority=k % 2)
        nl = k + 1
        if _NBUF <= nl < nc:
            stores[nl - _NBUF].wait()
            loads[nl].start(priority=nl % 2)
    for k in range(max(0, nc - _NBUF), nc):
        stores[k].wait()
    head.wait()


def kernel(data, x):
    n, d = data.shape
    flat = pl.pallas_call(
        _shift_body,
        in_specs=[
            pl.BlockSpec(memory_space=pl.ANY),
            pl.BlockSpec(memory_space=pl.ANY),
        ],
        out_specs=pl.BlockSpec(memory_space=pl.ANY),
        out_shape=jax.ShapeDtypeStruct((n * d,), data.dtype),
        scratch_shapes=[
            pltpu.VMEM((_NBUF, _BIG), jnp.float32),
            pltpu.SemaphoreType.DMA((_NBUF,)),
            pltpu.SemaphoreType.DMA((_NBUF,)),
            pltpu.SemaphoreType.DMA,
        ],
    )(data.reshape(-1), x)
    return flat.reshape(n, d)


# speedup vs baseline: 1.1354x; 1.1354x over previous
import jax
import jax.numpy as jnp
from jax.experimental import pallas as pl
from jax.experimental.pallas import tpu as pltpu

_BIG = 1 << 21
_SMALL = 1 << 19
_SIZES = [_BIG, _BIG, _BIG, _BIG, _BIG, _BIG, _BIG, _BIG - 128]
assert sum(_SIZES) == 128 * 131071
_NBUF = 5


def _shift_body(data_ref, x_ref, out_ref, *scr):
    bufs = scr[:_NBUF]
    lsems, ssems, hsem = scr[_NBUF:]
    nc = len(_SIZES)
    offs = [0]
    for s in _SIZES:
        offs.append(offs[-1] + s)

    def load(k):
        b = k % _NBUF
        return pltpu.make_async_copy(
            data_ref.at[pl.ds(offs[k], _SIZES[k])],
            bufs[b].at[pl.ds(0, _SIZES[k])],
            lsems.at[b],
        )

    def store(k):
        b = k % _NBUF
        return pltpu.make_async_copy(
            bufs[b].at[pl.ds(0, _SIZES[k])],
            out_ref.at[pl.ds(128 + offs[k], _SIZES[k])],
            ssems.at[b],
        )

    loads = [load(k) for k in range(nc)]
    stores = [store(k) for k in range(nc)]

    head = pltpu.make_async_copy(x_ref, out_ref.at[pl.ds(0, 128)], hsem)
    head.start()

    for k in range(min(_NBUF, nc)):
        loads[k].start(priority=k % 2)
    for k in range(nc):
        loads[k].wait()
        stores[k].start(priority=k % 2)
        nl = k + 1
        if _NBUF <= nl < nc:
            stores[nl - _NBUF].wait()
            loads[nl].start(priority=nl % 2)
    for k in range(max(0, nc - _NBUF), nc):
        stores[k].wait()
    head.wait()


def kernel(data, x):
    n, d = data.shape
    flat = pl.pallas_call(
        _shift_body,
        in_specs=[
            pl.BlockSpec(memory_space=pl.ANY),
            pl.BlockSpec(memory_space=pl.ANY),
        ],
        out_specs=pl.BlockSpec(memory_space=pl.ANY),
        out_shape=jax.ShapeDtypeStruct((n * d,), data.dtype),
        scratch_shapes=(
            [pltpu.VMEM((_BIG,), jnp.float32) for _ in range(_NBUF)]
            + [
                pltpu.SemaphoreType.DMA((_NBUF,)),
                pltpu.SemaphoreType.DMA((_NBUF,)),
                pltpu.SemaphoreType.DMA,
            ]
        ),
    )(data.reshape(-1), x)
    return flat.reshape(n, d)


# speedup vs baseline: 1.2002x; 1.0571x over previous
import jax
import jax.numpy as jnp
from jax.experimental import pallas as pl
from jax.experimental.pallas import tpu as pltpu

_BIG = 1 << 21
_SMALL = 1 << 19
_SIZES = [_BIG, _BIG, _BIG, _BIG, _BIG, _BIG, _BIG, _BIG - 128]
assert sum(_SIZES) == 128 * 131071
_NBUF = 7


def _shift_body(data_ref, x_ref, out_ref, *scr):
    bufs = scr[:_NBUF]
    lsems, ssems, hsem = scr[_NBUF:]
    nc = len(_SIZES)
    offs = [0]
    for s in _SIZES:
        offs.append(offs[-1] + s)

    def load(k):
        b = k % _NBUF
        return pltpu.make_async_copy(
            data_ref.at[pl.ds(offs[k], _SIZES[k])],
            bufs[b].at[pl.ds(0, _SIZES[k])],
            lsems.at[b],
        )

    def store(k):
        b = k % _NBUF
        return pltpu.make_async_copy(
            bufs[b].at[pl.ds(0, _SIZES[k])],
            out_ref.at[pl.ds(128 + offs[k], _SIZES[k])],
            ssems.at[b],
        )

    loads = [load(k) for k in range(nc)]
    stores = [store(k) for k in range(nc)]

    head = pltpu.make_async_copy(x_ref, out_ref.at[pl.ds(0, 128)], hsem)
    head.start()

    for k in range(min(_NBUF, nc)):
        loads[k].start(priority=k % 2)
    for k in range(nc):
        loads[k].wait()
        stores[k].start(priority=k % 2)
        nl = k + 1
        if _NBUF <= nl < nc:
            stores[nl - _NBUF].wait()
            loads[nl].start(priority=nl % 2)
    for k in range(max(0, nc - _NBUF), nc):
        stores[k].wait()
    head.wait()


def kernel(data, x):
    n, d = data.shape
    flat = pl.pallas_call(
        _shift_body,
        in_specs=[
            pl.BlockSpec(memory_space=pl.ANY),
            pl.BlockSpec(memory_space=pl.ANY),
        ],
        out_specs=pl.BlockSpec(memory_space=pl.ANY),
        out_shape=jax.ShapeDtypeStruct((n * d,), data.dtype),
        scratch_shapes=(
            [pltpu.VMEM((_BIG,), jnp.float32) for _ in range(_NBUF)]
            + [
                pltpu.SemaphoreType.DMA((_NBUF,)),
                pltpu.SemaphoreType.DMA((_NBUF,)),
                pltpu.SemaphoreType.DMA,
            ]
        ),
    )(data.reshape(-1), x)
    return flat.reshape(n, d)
